# baseline (speedup 1.0000x reference)
import jax
import jax.numpy as jnp
from jax.experimental import pallas as pl
from jax.experimental.pallas import tpu as pltpu

BETA = 0.2
GAMMA = 0.95


def _zmax_body(z_ref, o_ref):
    o_ref[...] = jnp.max(z_ref[...], axis=1, keepdims=True)


def _scatter_body(zmax_ref, src_ref, dst_ref, e_ref, accq_ref, acca_ref):
    i = pl.program_id(0)

    @pl.when(i == 0)
    def _init():
        accq_ref[...] = jnp.zeros_like(accq_ref)
        acca_ref[...] = jnp.zeros_like(acca_ref)

    nb = e_ref.shape[0]
    dcols = acca_ref.shape[1]

    def body(j, _):
        s = src_ref[0, 0, j]
        d = dst_ref[0, 0, j]
        c = GAMMA * zmax_ref[s >> 7, s & 127]
        e0 = e_ref[pl.ds(j, 1), 0:dcols]
        e1 = e_ref[pl.ds(j, 1), dcols:2 * dcols]
        accq_ref[pl.ds(d, 1), :] = accq_ref[pl.ds(d, 1), :] + (e0 + c * e1)
        acca_ref[pl.ds(d, 1), :] = acca_ref[pl.ds(d, 1), :] + e1
        return 0

    jax.lax.fori_loop(0, nb, body, 0, unroll=False)


def _combine_body(z_ref, accq_ref, acca_ref, o_ref):
    o_ref[...] = (BETA * z_ref[...]
                  + (1.0 - BETA) * accq_ref[...] / (acca_ref[...] + 1e-6))


def kernel(z, e, edge_index):
    n, d = z.shape
    en = e.shape[0]

    npad = ((n + 127) // 128) * 128
    z_pad = jnp.pad(z, ((0, npad - n), (0, 0)))

    eb = 2000
    assert en % eb == 0
    neb = en // eb

    src = edge_index[0].astype(jnp.int32).reshape(neb, 1, eb)
    dst = edge_index[1].astype(jnp.int32).reshape(neb, 1, eb)
    e2 = e.reshape(en, 2 * d)

    zmax = pl.pallas_call(
        _zmax_body,
        grid=(npad // 128,),
        in_specs=[pl.BlockSpec((128, d), lambda i: (i, 0))],
        out_specs=pl.BlockSpec((128, 1), lambda i: (i, 0)),
        out_shape=jax.ShapeDtypeStruct((npad, 1), jnp.float32),
    )(z_pad)

    zmax2 = zmax.reshape(npad // 128, 128)

    accq, acca = pl.pallas_call(
        _scatter_body,
        grid=(neb,),
        in_specs=[
            pl.BlockSpec(memory_space=pltpu.SMEM),
            pl.BlockSpec((1, 1, eb), lambda i: (i, 0, 0),
                         memory_space=pltpu.SMEM),
            pl.BlockSpec((1, 1, eb), lambda i: (i, 0, 0),
                         memory_space=pltpu.SMEM),
            pl.BlockSpec((eb, 2 * d), lambda i: (i, 0)),
        ],
        out_specs=[
            pl.BlockSpec((npad, d), lambda i: (0, 0)),
            pl.BlockSpec((npad, d), lambda i: (0, 0)),
        ],
        out_shape=[
            jax.ShapeDtypeStruct((npad, d), jnp.float32),
            jax.ShapeDtypeStruct((npad, d), jnp.float32),
        ],
    )(zmax2, src, dst, e2)

    out = pl.pallas_call(
        _combine_body,
        grid=(npad // 128,),
        in_specs=[
            pl.BlockSpec((128, d), lambda i: (i, 0)),
            pl.BlockSpec((128, d), lambda i: (i, 0)),
            pl.BlockSpec((128, d), lambda i: (i, 0)),
        ],
        out_specs=pl.BlockSpec((128, d), lambda i: (i, 0)),
        out_shape=jax.ShapeDtypeStruct((npad, d), jnp.float32),
    )(z_pad, accq, acca)

    return out[:n]


# speedup vs baseline: 1.3166x; 1.3166x over previous
import dataclasses
import functools

import jax
import jax.numpy as jnp
from jax import lax
from jax.experimental import pallas as pl
from jax.experimental.pallas import tpu as pltpu
from jax.experimental.pallas import tpu_sc as plsc

BETA = 0.2
GAMMA = 0.95

L = 16
NW = 32
NR = 160
NSWEEP = 2
K = 64
CAP = K + L


def _zmax_body(z_ref, o_ref):
    o_ref[...] = jnp.max(z_ref[...], axis=1, keepdims=True)


def _combine_body(z_ref, sp_ref, o_ref):
    d = z_ref.shape[1]
    sq = sp_ref[:, 0:d]
    sa = sp_ref[:, d:2 * d]
    o_ref[...] = BETA * z_ref[...] + (1.0 - BETA) * sq / (sa + 1e-6)


def _sc_params():
    mesh = plsc.VectorSubcoreMesh(core_axis_name="c", subcore_axis_name="s")
    cparams = pltpu.CompilerParams()
    if "needs_layout_passes" in pltpu.CompilerParams.__dataclass_fields__:
        cparams = dataclasses.replace(cparams, needs_layout_passes=False)
    return mesh, cparams


def _make_sc_coeff(npad, enp):
    ew = enp // NW
    mesh, cparams = _sc_params()

    @functools.partial(
        pl.kernel,
        mesh=mesh,
        compiler_params=cparams,
        out_type=jax.ShapeDtypeStruct((enp,), jnp.float32),
        scratch_types=[
            pltpu.VMEM((npad,), jnp.float32),
            pltpu.VMEM((ew,), jnp.int32),
            pltpu.VMEM((ew,), jnp.float32),
        ],
    )
    def sc_coeff(zmax_hbm, src_hbm, call_hbm, zmaxv, srcb, cb):
        wid = lax.axis_index("s") * 2 + lax.axis_index("c")
        pltpu.sync_copy(zmax_hbm, zmaxv)
        pltpu.sync_copy(src_hbm.at[pl.ds(wid * ew, ew)], srcb)

        @pl.loop(0, ew // L)
        def _grp(g):
            srcv = srcb[pl.ds(g * L, L)]
            cb[pl.ds(g * L, L)] = GAMMA * plsc.load_gather(zmaxv, [srcv])

        pltpu.sync_copy(cb, call_hbm.at[pl.ds(wid * ew, ew)])

    return sc_coeff


def _make_sc_scatter(npad, en, enp, d):
    s = 4000
    nch = en // s
    ngrp = s // L
    d2 = 2 * d
    nrp = NR + 8
    mesh, cparams = _sc_params()

    @functools.partial(
        pl.kernel,
        mesh=mesh,
        compiler_params=cparams,
        out_type=jax.ShapeDtypeStruct((npad, d2), jnp.float32),
        scratch_types=[
            pltpu.VMEM((s,), jnp.int32),
            pltpu.VMEM((s,), jnp.float32),
            pltpu.VMEM((CAP,), jnp.int32),
            pltpu.VMEM((CAP,), jnp.int32),
            pltpu.VMEM((CAP,), jnp.float32),
            pltpu.VMEM((K, d2), jnp.float32),
            pltpu.VMEM((nrp, d2), jnp.float32),
        ],
    )
    def sc_scatter(dst_hbm, call_hbm, e2_hbm, sump_hbm,
                   dstbuf, cchunk, eidbuf, ldstbuf, cbuf, epair, acc):
        wid = lax.axis_index("s") * 2 + lax.axis_index("c")
        iota = lax.iota(jnp.int32, L)

        def fire(f):
            pltpu.sync_copy(e2_hbm.at[eidbuf.at[pl.ds(0, K)]], epair)

            @pl.loop(0, K)
            def _edge(h):
                cfull = cbuf[pl.ds((h >> 4) * L, L)]
                ldv = ldstbuf[pl.ds((h >> 4) * L, L)]
                lanev = jnp.full((L,), h & (L - 1), jnp.int32)
                dnums = lax.GatherDimensionNumbers(
                    offset_dims=(), collapsed_slice_dims=(0,),
                    start_index_map=(0,))
                csp = lax.gather(
                    cfull, lanev[:, None], dnums, (1,),
                    mode=lax.GatherScatterMode.PROMISE_IN_BOUNDS)
                r = jnp.max(lax.gather(
                    ldv, lanev[:, None], dnums, (1,),
                    mode=lax.GatherScatterMode.PROMISE_IN_BOUNDS))
                for kk in range(d // L):
                    e1c = epair[h, pl.ds(d + kk * L, L)]
                    plsc.addupdate(acc.at[r, pl.ds(kk * L, L)],
                                   epair[h, pl.ds(kk * L, L)] + e1c * csp)
                    plsc.addupdate(acc.at[r, pl.ds(d + kk * L, L)], e1c)

            eidbuf[pl.ds(0, L)] = eidbuf[pl.ds(K, L)]
            ldstbuf[pl.ds(0, L)] = ldstbuf[pl.ds(K, L)]
            cbuf[pl.ds(0, L)] = cbuf[pl.ds(K, L)]
            return f - K

        @pl.loop(0, NSWEEP)
        def _sweep(t):
            base = (t * NW + wid) * NR

            @pl.loop(0, nrp)
            def _zr(r):
                @pl.loop(0, d2 // L)
                def _zc(kk):
                    acc[r, pl.ds(kk * L, L)] = jnp.zeros((L,), jnp.float32)

            def _chunk(ch, fill):
                ebase = ch * s
                pltpu.sync_copy(dst_hbm.at[pl.ds(ebase, s)], dstbuf)
                pltpu.sync_copy(call_hbm.at[pl.ds(ebase, s)], cchunk)

                def _grp(g, f):
                    dstv = dstbuf[pl.ds(g * L, L)]
                    ldstv = dstv - base
                    m = (ldstv >= 0) & (ldstv < NR)
                    cnt = jnp.max(plsc.all_reduce_population_count(m))

                    def hit(f2):
                        posv = f2 + plsc.cumsum(m.astype(jnp.int32)) - 1
                        cv = cchunk[pl.ds(g * L, L)]
                        eidv = ebase + g * L + iota
                        plsc.store_scatter(eidbuf, [posv], eidv, mask=m)
                        plsc.store_scatter(ldstbuf, [posv], ldstv, mask=m)
                        plsc.store_scatter(cbuf, [posv], cv, mask=m)
                        f3 = f2 + cnt
                        return lax.cond(f3 >= K, fire, lambda x: x, f3)

                    return lax.cond(cnt > 0, hit, lambda x: x, f)

                return lax.fori_loop(0, ngrp, _grp, fill)

            fill = lax.fori_loop(0, nch, _chunk, jnp.int32(0))

            for gi in range(CAP // L):
                p = fill + gi * L + iota
                pm = p < CAP
                plsc.store_scatter(eidbuf, [p], p, mask=pm)
                plsc.store_scatter(ldstbuf, [p], NR + (iota & 7), mask=pm)
                plsc.store_scatter(cbuf, [p],
                                   jnp.zeros((L,), jnp.float32), mask=pm)
            lax.cond(fill > 0, fire, lambda x: x, fill)

            pltpu.sync_copy(acc.at[pl.ds(0, NR)],
                            sump_hbm.at[pl.ds(base, NR)])

    return sc_scatter


def kernel(z, e, edge_index):
    n, d = z.shape
    en = e.shape[0]

    npad = NW * NR * NSWEEP
    z_pad = jnp.pad(z, ((0, npad - n), (0, 0)))

    enp = ((en + (L * NW) - 1) // (L * NW)) * (L * NW)
    src = jnp.pad(edge_index[0].astype(jnp.int32), (0, enp - en))
    dst = edge_index[1].astype(jnp.int32)
    e2 = e.reshape(en, 2 * d)

    zmax = pl.pallas_call(
        _zmax_body,
        grid=(npad // 64,),
        in_specs=[pl.BlockSpec((64, d), lambda i: (i, 0))],
        out_specs=pl.BlockSpec((64, 1), lambda i: (i, 0)),
        out_shape=jax.ShapeDtypeStruct((npad, 1), jnp.float32),
    )(z_pad)

    call = _make_sc_coeff(npad, enp)(zmax.reshape(npad), src)
    sump = _make_sc_scatter(npad, en, enp, d)(dst, call, e2)

    out = pl.pallas_call(
        _combine_body,
        grid=(npad // 64,),
        in_specs=[
            pl.BlockSpec((64, d), lambda i: (i, 0)),
            pl.BlockSpec((64, 2 * d), lambda i: (i, 0)),
        ],
        out_specs=pl.BlockSpec((64, d), lambda i: (i, 0)),
        out_shape=jax.ShapeDtypeStruct((npad, d), jnp.float32),
    )(z_pad, sump)

    return out[:n]


# speedup vs baseline: 1.5716x; 1.1937x over previous
import dataclasses
import functools

import jax
import jax.numpy as jnp
from jax import lax
from jax.experimental import pallas as pl
from jax.experimental.pallas import tpu as pltpu
from jax.experimental.pallas import tpu_sc as plsc

BETA = 0.2
GAMMA = 0.95

L = 16
NW = 32
NR = 160
NSWEEP = 2
K = 64
CAP = K + L


def _zmax_body(z_ref, o_ref):
    o_ref[...] = jnp.max(z_ref[...], axis=1, keepdims=True)


def _combine_body(z_ref, sp_ref, o_ref):
    d = z_ref.shape[1]
    sq = sp_ref[:, 0:d]
    sa = sp_ref[:, d:2 * d]
    o_ref[...] = BETA * z_ref[...] + (1.0 - BETA) * sq / (sa + 1e-6)


def _sc_params():
    mesh = plsc.VectorSubcoreMesh(core_axis_name="c", subcore_axis_name="s")
    cparams = pltpu.CompilerParams()
    if "needs_layout_passes" in pltpu.CompilerParams.__dataclass_fields__:
        cparams = dataclasses.replace(cparams, needs_layout_passes=False)
    return mesh, cparams


def _make_sc_coeff(npad, enp):
    ew = enp // NW
    mesh, cparams = _sc_params()

    @functools.partial(
        pl.kernel,
        mesh=mesh,
        compiler_params=cparams,
        out_type=jax.ShapeDtypeStruct((enp,), jnp.float32),
        scratch_types=[
            pltpu.VMEM((npad,), jnp.float32),
            pltpu.VMEM((ew,), jnp.int32),
            pltpu.VMEM((ew,), jnp.float32),
        ],
    )
    def sc_coeff(zmax_hbm, src_hbm, call_hbm, zmaxv, srcb, cb):
        wid = lax.axis_index("s") * 2 + lax.axis_index("c")
        pltpu.sync_copy(zmax_hbm, zmaxv)
        pltpu.sync_copy(src_hbm.at[pl.ds(wid * ew, ew)], srcb)

        @pl.loop(0, ew // L)
        def _grp(g):
            srcv = srcb[pl.ds(g * L, L)]
            cb[pl.ds(g * L, L)] = GAMMA * plsc.load_gather(zmaxv, [srcv])

        pltpu.sync_copy(cb, call_hbm.at[pl.ds(wid * ew, ew)])

    return sc_coeff


def _make_sc_scatter(npad, en, enp, d):
    s = 4000
    nch = en // s
    ngrp = s // L
    d2 = 2 * d
    nrp = NR + 8
    mesh, cparams = _sc_params()

    @functools.partial(
        pl.kernel,
        mesh=mesh,
        compiler_params=cparams,
        out_type=jax.ShapeDtypeStruct((npad, d2), jnp.float32),
        scratch_types=[
            pltpu.VMEM((s,), jnp.int32),
            pltpu.VMEM((s,), jnp.float32),
            pltpu.VMEM((CAP,), jnp.int32),
            pltpu.VMEM((CAP,), jnp.int32),
            pltpu.VMEM((CAP,), jnp.float32),
            pltpu.VMEM((K, d2), jnp.float32),
            pltpu.VMEM((nrp, d2), jnp.float32),
        ],
    )
    def sc_scatter(dst_hbm, call_hbm, e2_hbm, sump_hbm,
                   dstbuf, cchunk, eidbuf, ldstbuf, cbuf, epair, acc):
        wid = lax.axis_index("s") * 2 + lax.axis_index("c")
        iota = lax.iota(jnp.int32, L)

        def fire(f):
            pltpu.sync_copy(e2_hbm.at[eidbuf.at[pl.ds(0, K)]], epair)

            @plsc.parallel_loop(0, K, unroll=4)
            def _edge(h):
                cfull = cbuf[pl.ds((h >> 4) * L, L)]
                ldv = ldstbuf[pl.ds((h >> 4) * L, L)]
                lanev = jnp.full((L,), h & (L - 1), jnp.int32)
                dnums = lax.GatherDimensionNumbers(
                    offset_dims=(), collapsed_slice_dims=(0,),
                    start_index_map=(0,))
                csp = lax.gather(
                    cfull, lanev[:, None], dnums, (1,),
                    mode=lax.GatherScatterMode.PROMISE_IN_BOUNDS)
                r = jnp.max(lax.gather(
                    ldv, lanev[:, None], dnums, (1,),
                    mode=lax.GatherScatterMode.PROMISE_IN_BOUNDS))
                for kk in range(d // L):
                    e1c = epair[h, pl.ds(d + kk * L, L)]
                    plsc.addupdate(acc.at[r, pl.ds(kk * L, L)],
                                   epair[h, pl.ds(kk * L, L)] + e1c * csp)
                    plsc.addupdate(acc.at[r, pl.ds(d + kk * L, L)], e1c)

            eidbuf[pl.ds(0, L)] = eidbuf[pl.ds(K, L)]
            ldstbuf[pl.ds(0, L)] = ldstbuf[pl.ds(K, L)]
            cbuf[pl.ds(0, L)] = cbuf[pl.ds(K, L)]
            return f - K

        @pl.loop(0, NSWEEP)
        def _sweep(t):
            base = (t * NW + wid) * NR

            @pl.loop(0, nrp)
            def _zr(r):
                @pl.loop(0, d2 // L)
                def _zc(kk):
                    acc[r, pl.ds(kk * L, L)] = jnp.zeros((L,), jnp.float32)

            def _chunk(ch, fill):
                ebase = ch * s
                pltpu.sync_copy(dst_hbm.at[pl.ds(ebase, s)], dstbuf)
                pltpu.sync_copy(call_hbm.at[pl.ds(ebase, s)], cchunk)

                def _grp(g, f):
                    dstv = dstbuf[pl.ds(g * L, L)]
                    ldstv = dstv - base
                    m = (ldstv >= 0) & (ldstv < NR)
                    cnt = jnp.max(plsc.all_reduce_population_count(m))

                    def hit(f2):
                        posv = f2 + plsc.cumsum(m.astype(jnp.int32)) - 1
                        cv = cchunk[pl.ds(g * L, L)]
                        eidv = ebase + g * L + iota
                        plsc.store_scatter(eidbuf, [posv], eidv, mask=m)
                        plsc.store_scatter(ldstbuf, [posv], ldstv, mask=m)
                        plsc.store_scatter(cbuf, [posv], cv, mask=m)
                        f3 = f2 + cnt
                        return lax.cond(f3 >= K, fire, lambda x: x, f3)

                    return lax.cond(cnt > 0, hit, lambda x: x, f)

                return lax.fori_loop(0, ngrp, _grp, fill)

            fill = lax.fori_loop(0, nch, _chunk, jnp.int32(0))

            for gi in range(CAP // L):
                p = fill + gi * L + iota
                pm = p < CAP
                plsc.store_scatter(eidbuf, [p], p, mask=pm)
                plsc.store_scatter(ldstbuf, [p], NR + (iota & 7), mask=pm)
                plsc.store_scatter(cbuf, [p],
                                   jnp.zeros((L,), jnp.float32), mask=pm)
            lax.cond(fill > 0, fire, lambda x: x, fill)

            pltpu.sync_copy(acc.at[pl.ds(0, NR)],
                            sump_hbm.at[pl.ds(base, NR)])

    return sc_scatter


def kernel(z, e, edge_index):
    n, d = z.shape
    en = e.shape[0]

    npad = NW * NR * NSWEEP
    z_pad = jnp.pad(z, ((0, npad - n), (0, 0)))

    enp = ((en + (L * NW) - 1) // (L * NW)) * (L * NW)
    src = jnp.pad(edge_index[0].astype(jnp.int32), (0, enp - en))
    dst = edge_index[1].astype(jnp.int32)
    e2 = e.reshape(en, 2 * d)

    zmax = pl.pallas_call(
        _zmax_body,
        grid=(npad // 64,),
        in_specs=[pl.BlockSpec((64, d), lambda i: (i, 0))],
        out_specs=pl.BlockSpec((64, 1), lambda i: (i, 0)),
        out_shape=jax.ShapeDtypeStruct((npad, 1), jnp.float32),
    )(z_pad)

    call = _make_sc_coeff(npad, enp)(zmax.reshape(npad), src)
    sump = _make_sc_scatter(npad, en, enp, d)(dst, call, e2)

    out = pl.pallas_call(
        _combine_body,
        grid=(npad // 64,),
        in_specs=[
            pl.BlockSpec((64, d), lambda i: (i, 0)),
            pl.BlockSpec((64, 2 * d), lambda i: (i, 0)),
        ],
        out_specs=pl.BlockSpec((64, d), lambda i: (i, 0)),
        out_shape=jax.ShapeDtypeStruct((npad, d), jnp.float32),
    )(z_pad, sump)

    return out[:n]
